# Initial kernel scaffold; baseline (speedup 1.0000x reference)
import jax
import jax.numpy as jnp
from jax.experimental import pallas as pl


def kernel(x, edge_index, W, b, a):
    raise NotImplementedError("write your pallas kernel here")


# speedup vs baseline: 9.0648x; 9.0648x over previous
import functools
import jax
import jax.numpy as jnp
from jax import lax
from jax.experimental import pallas as pl
from jax.experimental.pallas import tpu as pltpu
from jax.experimental.pallas import tpu_sc as plsc

N = 10000
NP = 10240
D = 128
DA = 144
H = 256
E = 320000
NSC = 2
NT = 16
CH = 80
ROWS_T = NP // NT
_SC_MESH = plsc.VectorSubcoreMesh(core_axis_name="c", subcore_axis_name="s", num_cores=NSC, num_subcores=NT)
_SC_PARAMS = pltpu.CompilerParams(use_tc_tiling_on_sc=False)


@functools.partial(
    pl.kernel,
    out_type=jax.ShapeDtypeStruct((NSC, NP, 16), jnp.float32),
    mesh=_SC_MESH,
    compiler_params=_SC_PARAMS,
    scratch_types=[
        pltpu.VMEM_SHARED((NP, 16), jnp.float32),
        pltpu.VMEM((125, CH), jnp.int32),
        pltpu.VMEM((CH, 16), jnp.float32),
        pltpu.VMEM((CH, 16), jnp.float32),
    ],
)
def _deg_kernel(dst_hbm, deg_hbm, deg_sh, dst_v, ones_v, zeros_v):
    c = lax.axis_index("c")
    s = lax.axis_index("s")
    wid = c * NT + s

    def _fill(i, carry):
        ones_v[i, :] = jnp.full((16,), 1.0, jnp.float32)
        zeros_v[i, :] = jnp.zeros((16,), jnp.float32)
        return carry
    lax.fori_loop(0, CH, _fill, 0)

    def _zero(j, carry):
        pltpu.sync_copy(zeros_v, deg_sh.at[pl.ds(s * ROWS_T + j * CH, CH)])
        return carry
    lax.fori_loop(0, ROWS_T // CH, _zero, 0)
    plsc.subcore_barrier()

    pltpu.sync_copy(dst_hbm.at[wid], dst_v)

    def _acc(j, carry):
        pltpu.sync_copy(ones_v, deg_sh.at[dst_v.at[j]], add=True)
        return carry
    lax.fori_loop(0, 125, _acc, 0)
    plsc.subcore_barrier()

    pltpu.sync_copy(deg_sh.at[pl.ds(s * ROWS_T, ROWS_T)],
                    deg_hbm.at[c, pl.ds(s * ROWS_T, ROWS_T)])


def _scale_body(xa_ref, d_ref, y_ref, r_ref):
    deg = d_ref[0, :, :] + d_ref[1, :, :]
    r = lax.rsqrt(jnp.maximum(deg, 1.0))
    r_ref[:, :] = r
    y_ref[:, :] = xa_ref[:, :] * r[:, 0:1]


def _scale_call(x_aug, deg):
    bn = 1024
    grid = NP // bn
    return pl.pallas_call(
        _scale_body,
        grid=(grid,),
        in_specs=[
            pl.BlockSpec((bn, DA), lambda i: (i, 0)),
            pl.BlockSpec((NSC, bn, 16), lambda i: (0, i, 0)),
        ],
        out_specs=[
            pl.BlockSpec((bn, DA), lambda i: (i, 0)),
            pl.BlockSpec((bn, 16), lambda i: (i, 0)),
        ],
        out_shape=[
            jax.ShapeDtypeStruct((NP, DA), jnp.float32),
            jax.ShapeDtypeStruct((NP, 16), jnp.float32),
        ],
    )(x_aug, deg)


@functools.partial(
    pl.kernel,
    out_type=[
        jax.ShapeDtypeStruct((NSC, NP, DA), jnp.float32),
        jax.ShapeDtypeStruct((NSC, NP, DA), jnp.float32),
    ],
    mesh=_SC_MESH,
    compiler_params=_SC_PARAMS,
    scratch_types=[
        pltpu.VMEM_SHARED((NP, DA), jnp.float32),
        pltpu.VMEM((CH,), jnp.int32),
        pltpu.VMEM((CH,), jnp.int32),
        pltpu.VMEM((CH, DA), jnp.float32),
        pltpu.VMEM((CH, DA), jnp.float32),
        pltpu.VMEM((8, CH), jnp.int32),
        pltpu.VMEM((CH, 16), jnp.float32),
        pltpu.SemaphoreType.DMA,
    ],
)
def _agg_kernel(y_hbm, xa_hbm, rdeg_hbm, perm_hbm, src_hbm, dst_hbm,
                acc_hbm, tab_hbm,
                acc_sh, src_i, dst_i, rows_v, zeros_v, pidx_v, rdeg_v, sem):
    c = lax.axis_index("c")
    s = lax.axis_index("s")
    row0 = s * ROWS_T

    def _fill(i, carry):
        def _fill16(k, inner):
            zeros_v[i, pl.ds(k * 16, 16)] = jnp.zeros((16,), jnp.float32)
            return inner
        lax.fori_loop(0, DA // 16, _fill16, 0)
        return carry
    lax.fori_loop(0, CH, _fill, 0)

    def _zero(j, carry):
        pltpu.sync_copy(zeros_v, acc_sh.at[pl.ds(row0 + j * CH, CH)])
        return carry
    lax.fori_loop(0, ROWS_T // CH, _zero, 0)

    @pl.when(c == 0)
    def _():
        pltpu.sync_copy(y_hbm.at[pl.ds(row0, ROWS_T)],
                        tab_hbm.at[c, pl.ds(row0, ROWS_T)])

    @pl.when(c == 1)
    def _():
        pltpu.sync_copy(perm_hbm.at[s], pidx_v)

        def _gather(j, carry):
            pltpu.async_copy(xa_hbm.at[pidx_v.at[j]], rows_v, sem).wait()
            pltpu.sync_copy(rdeg_hbm.at[pl.ds(row0 + j * CH, CH)], rdeg_v)

            def _scale_row(r, c1):
                rv = rdeg_v[r, :]

                def _scale16(k, c2):
                    rows_v[r, pl.ds(k * 16, 16)] = rows_v[r, pl.ds(k * 16, 16)] * rv
                    return c2
                lax.fori_loop(0, DA // 16, _scale16, 0)
                return c1
            lax.fori_loop(0, CH, _scale_row, 0)
            pltpu.sync_copy(rows_v, tab_hbm.at[c, pl.ds(row0 + j * CH, CH)])
            return carry
        lax.fori_loop(0, ROWS_T // CH, _gather, 0)

    plsc.subcore_barrier()

    table = tab_hbm.at[c]

    def _chunk(j, carry):
        pltpu.sync_copy(src_hbm.at[s, j], src_i)
        pltpu.sync_copy(dst_hbm.at[s, j], dst_i)
        pltpu.async_copy(table.at[src_i], rows_v, sem).wait()
        pltpu.sync_copy(rows_v, acc_sh.at[dst_i], add=True)
        return carry
    lax.fori_loop(0, 250, _chunk, 0)

    plsc.subcore_barrier()
    pltpu.sync_copy(acc_sh.at[pl.ds(row0, ROWS_T)],
                    acc_hbm.at[c, pl.ds(row0, ROWS_T)])


def _out_body(accp_ref, accn_ref, r_ref, wb_ref, a_ref, pz_ref, nz_ref, sm_ref):
    i = pl.program_id(0)
    wb = wb_ref[:, :]
    r = r_ref[:, 0:1]
    slope = a_ref[0, 0]

    hp = jnp.dot(accp_ref[0, :, :], wb, preferred_element_type=jnp.float32) * r
    pz = jnp.where(hp > 0, hp, slope * hp)
    pz_ref[:, :] = pz

    hn = jnp.dot(accn_ref[0, :, :], wb, preferred_element_type=jnp.float32) * r
    nz = jnp.where(hn > 0, hn, slope * hn)
    nz_ref[:, :] = nz

    @pl.when(i == 0)
    def _():
        sm_ref[:, :] = jnp.zeros((8, H), jnp.float32)

    sm_ref[0:1, :] += jnp.sum(pz, axis=0, keepdims=True)

    @pl.when(i == pl.num_programs(0) - 1)
    def _():
        sm_ref[0:1, :] = jax.nn.sigmoid(sm_ref[0:1, :] * (1.0 / N))


def _out_call(acc, rdeg, wb, a2d):
    bn = 1024
    grid = NP // bn
    return pl.pallas_call(
        _out_body,
        grid=(grid,),
        in_specs=[
            pl.BlockSpec((1, bn, DA), lambda i: (0, i, 0)),
            pl.BlockSpec((1, bn, DA), lambda i: (1, i, 0)),
            pl.BlockSpec((bn, 16), lambda i: (i, 0)),
            pl.BlockSpec((DA, H), lambda i: (0, 0)),
            pl.BlockSpec((1, 1), lambda i: (0, 0)),
        ],
        out_specs=[
            pl.BlockSpec((bn, H), lambda i: (i, 0)),
            pl.BlockSpec((bn, H), lambda i: (i, 0)),
            pl.BlockSpec((8, H), lambda i: (0, 0)),
        ],
        out_shape=[
            jax.ShapeDtypeStruct((NP, H), jnp.float32),
            jax.ShapeDtypeStruct((NP, H), jnp.float32),
            jax.ShapeDtypeStruct((8, H), jnp.float32),
        ],
    )(acc, acc, rdeg, wb, a2d)


@jax.jit
def kernel(x, edge_index, W, b, a):
    src = edge_index[0].astype(jnp.int32)
    dst = edge_index[1].astype(jnp.int32)

    x_aug = jnp.concatenate(
        [x, jnp.ones((N, 1), jnp.float32), jnp.zeros((N, 15), jnp.float32)], axis=1)
    x_aug = jnp.pad(x_aug, ((0, NP - N), (0, 0)))
    wb = jnp.concatenate([W, b[None, :], jnp.zeros((15, H), jnp.float32)], axis=0)
    perm = jax.random.permutation(jax.random.key(42), N).astype(jnp.int32)
    perm_pad = jnp.pad(perm, (0, NP - N)).reshape(NT, (NP // NT) // CH, CH)
    dst_a = dst.reshape(NSC * NT, (E // (NSC * NT)) // CH, CH)
    src_c = src.reshape(NT, (E // NT) // CH, CH)
    dst_c = dst.reshape(NT, (E // NT) // CH, CH)
    a2d = a.reshape(1, 1)

    deg = _deg_kernel(dst_a)
    y, rdeg = _scale_call(x_aug, deg)
    acc, _unused = _agg_kernel(y, x_aug, rdeg, perm_pad, src_c, dst_c)
    pz, nz, sm = _out_call(acc, rdeg, wb, a2d)

    return (pz[:N], nz[:N], sm[0, :])


# speedup vs baseline: 12.4374x; 1.3721x over previous
import functools
import jax
import jax.numpy as jnp
from jax import lax
from jax.experimental import pallas as pl
from jax.experimental.pallas import tpu as pltpu
from jax.experimental.pallas import tpu_sc as plsc

N = 10000
NP = 10240
D = 128
DA = 144
H = 256
E = 320000
NSC = 2
NT = 16
CH = 80
ROWS_T = NP // NT
_SC_MESH = plsc.VectorSubcoreMesh(core_axis_name="c", subcore_axis_name="s", num_cores=NSC, num_subcores=NT)
_SC_PARAMS = pltpu.CompilerParams(use_tc_tiling_on_sc=False)


@functools.partial(
    pl.kernel,
    out_type=jax.ShapeDtypeStruct((NSC, NP, 16), jnp.float32),
    mesh=_SC_MESH,
    compiler_params=_SC_PARAMS,
    scratch_types=[
        pltpu.VMEM_SHARED((NP, 16), jnp.float32),
        pltpu.VMEM((125, CH), jnp.int32),
        pltpu.VMEM((CH, 16), jnp.float32),
        pltpu.VMEM((CH, 16), jnp.float32),
    ],
)
def _deg_kernel(dst_hbm, deg_hbm, deg_sh, dst_v, ones_v, zeros_v):
    c = lax.axis_index("c")
    s = lax.axis_index("s")
    wid = c * NT + s

    def _fill(i, carry):
        ones_v[i, :] = jnp.full((16,), 1.0, jnp.float32)
        zeros_v[i, :] = jnp.zeros((16,), jnp.float32)
        return carry
    lax.fori_loop(0, CH, _fill, 0)

    def _zero(j, carry):
        pltpu.sync_copy(zeros_v, deg_sh.at[pl.ds(s * ROWS_T + j * CH, CH)])
        return carry
    lax.fori_loop(0, ROWS_T // CH, _zero, 0)
    plsc.subcore_barrier()

    pltpu.sync_copy(dst_hbm.at[wid], dst_v)

    def _acc(j, carry):
        pltpu.sync_copy(ones_v, deg_sh.at[dst_v.at[j]], add=True)
        return carry
    lax.fori_loop(0, 125, _acc, 0)
    plsc.subcore_barrier()

    pltpu.sync_copy(deg_sh.at[pl.ds(s * ROWS_T, ROWS_T)],
                    deg_hbm.at[c, pl.ds(s * ROWS_T, ROWS_T)])


def _scale_body(xa_ref, d_ref, y_ref, r_ref):
    deg = d_ref[0, :, :] + d_ref[1, :, :]
    r = lax.rsqrt(jnp.maximum(deg, 1.0))
    r_ref[:, :] = r
    y_ref[:, :] = xa_ref[:, :] * r[:, 0:1]


def _scale_call(x_aug, deg):
    bn = 1024
    grid = NP // bn
    return pl.pallas_call(
        _scale_body,
        grid=(grid,),
        in_specs=[
            pl.BlockSpec((bn, DA), lambda i: (i, 0)),
            pl.BlockSpec((NSC, bn, 16), lambda i: (0, i, 0)),
        ],
        out_specs=[
            pl.BlockSpec((bn, DA), lambda i: (i, 0)),
            pl.BlockSpec((bn, 16), lambda i: (i, 0)),
        ],
        out_shape=[
            jax.ShapeDtypeStruct((NP, DA), jnp.float32),
            jax.ShapeDtypeStruct((NP, 16), jnp.float32),
        ],
    )(x_aug, deg)


@functools.partial(
    pl.kernel,
    out_type=[
        jax.ShapeDtypeStruct((NSC, NP, DA), jnp.float32),
        jax.ShapeDtypeStruct((NSC, NP, DA), jnp.float32),
    ],
    mesh=_SC_MESH,
    compiler_params=_SC_PARAMS,
    scratch_types=[
        pltpu.VMEM_SHARED((NP, DA), jnp.float32),
        pltpu.VMEM((CH,), jnp.int32),
        pltpu.VMEM((CH,), jnp.int32),
        pltpu.VMEM((CH,), jnp.int32),
        pltpu.VMEM((CH,), jnp.int32),
        pltpu.VMEM((CH, DA), jnp.float32),
        pltpu.VMEM((CH, DA), jnp.float32),
        pltpu.VMEM((8, CH), jnp.int32),
        pltpu.VMEM((CH, 16), jnp.float32),
        pltpu.SemaphoreType.DMA,
        pltpu.SemaphoreType.DMA,
    ],
)
def _agg_kernel(y_hbm, xa_hbm, rdeg_hbm, perm_hbm, src_hbm, dst_hbm,
                acc_hbm, tab_hbm,
                acc_sh, src_a, dst_a, src_b, dst_b, rows_a, rows_b,
                pidx_v, rdeg_v, sem_a, sem_b):
    c = lax.axis_index("c")
    s = lax.axis_index("s")
    row0 = s * ROWS_T

    def _fill(i, carry):
        def _fill16(k, inner):
            rows_b[i, pl.ds(k * 16, 16)] = jnp.zeros((16,), jnp.float32)
            return inner
        lax.fori_loop(0, DA // 16, _fill16, 0)
        return carry
    lax.fori_loop(0, CH, _fill, 0)

    def _zero(j, carry):
        pltpu.sync_copy(rows_b, acc_sh.at[pl.ds(row0 + j * CH, CH)])
        return carry
    lax.fori_loop(0, ROWS_T // CH, _zero, 0)

    @pl.when(c == 0)
    def _():
        pltpu.sync_copy(y_hbm.at[pl.ds(row0, ROWS_T)],
                        tab_hbm.at[c, pl.ds(row0, ROWS_T)])

    @pl.when(c == 1)
    def _():
        pltpu.sync_copy(perm_hbm.at[s], pidx_v)

        def _gather(j, carry):
            pltpu.async_copy(xa_hbm.at[pidx_v.at[j]], rows_a, sem_a).wait()
            pltpu.sync_copy(rdeg_hbm.at[pl.ds(row0 + j * CH, CH)], rdeg_v)

            def _scale_row(r, c1):
                rv = rdeg_v[r, :]

                def _scale16(k, c2):
                    rows_a[r, pl.ds(k * 16, 16)] = rows_a[r, pl.ds(k * 16, 16)] * rv
                    return c2
                lax.fori_loop(0, DA // 16, _scale16, 0)
                return c1
            lax.fori_loop(0, CH, _scale_row, 0)
            pltpu.sync_copy(rows_a, tab_hbm.at[c, pl.ds(row0 + j * CH, CH)])
            return carry
        lax.fori_loop(0, ROWS_T // CH, _gather, 0)

    plsc.subcore_barrier()

    table = tab_hbm.at[c]
    NCHUNK = (E // NT) // CH

    def _fetch_a(j):
        pltpu.sync_copy(src_hbm.at[s, j], src_a)
        pltpu.sync_copy(dst_hbm.at[s, j], dst_a)
        pltpu.async_copy(table.at[src_a], rows_a, sem_a)

    def _fetch_b(j):
        pltpu.sync_copy(src_hbm.at[s, j], src_b)
        pltpu.sync_copy(dst_hbm.at[s, j], dst_b)
        pltpu.async_copy(table.at[src_b], rows_b, sem_b)

    _fetch_a(0)

    def _pair(t, carry):
        j0 = 2 * t
        _fetch_b(j0 + 1)
        pltpu.make_async_copy(table.at[src_a], rows_a, sem_a).wait()
        pltpu.sync_copy(rows_a, acc_sh.at[dst_a], add=True)

        @pl.when(t < NCHUNK // 2 - 1)
        def _():
            _fetch_a(j0 + 2)

        pltpu.make_async_copy(table.at[src_b], rows_b, sem_b).wait()
        pltpu.sync_copy(rows_b, acc_sh.at[dst_b], add=True)
        return carry
    lax.fori_loop(0, NCHUNK // 2, _pair, 0)

    plsc.subcore_barrier()
    pltpu.sync_copy(acc_sh.at[pl.ds(row0, ROWS_T)],
                    acc_hbm.at[c, pl.ds(row0, ROWS_T)])


def _out_body(accp_ref, accn_ref, r_ref, wb_ref, a_ref, pz_ref, nz_ref, sm_ref):
    i = pl.program_id(0)
    wb = wb_ref[:, :]
    r = r_ref[:, 0:1]
    slope = a_ref[0, 0]

    hp = jnp.dot(accp_ref[0, :, :], wb, preferred_element_type=jnp.float32) * r
    pz = jnp.where(hp > 0, hp, slope * hp)
    pz_ref[:, :] = pz

    hn = jnp.dot(accn_ref[0, :, :], wb, preferred_element_type=jnp.float32) * r
    nz = jnp.where(hn > 0, hn, slope * hn)
    nz_ref[:, :] = nz

    @pl.when(i == 0)
    def _():
        sm_ref[:, :] = jnp.zeros((8, H), jnp.float32)

    sm_ref[0:1, :] += jnp.sum(pz, axis=0, keepdims=True)

    @pl.when(i == pl.num_programs(0) - 1)
    def _():
        sm_ref[0:1, :] = jax.nn.sigmoid(sm_ref[0:1, :] * (1.0 / N))


def _out_call(acc, rdeg, wb, a2d):
    bn = 1024
    grid = NP // bn
    return pl.pallas_call(
        _out_body,
        grid=(grid,),
        in_specs=[
            pl.BlockSpec((1, bn, DA), lambda i: (0, i, 0)),
            pl.BlockSpec((1, bn, DA), lambda i: (1, i, 0)),
            pl.BlockSpec((bn, 16), lambda i: (i, 0)),
            pl.BlockSpec((DA, H), lambda i: (0, 0)),
            pl.BlockSpec((1, 1), lambda i: (0, 0)),
        ],
        out_specs=[
            pl.BlockSpec((bn, H), lambda i: (i, 0)),
            pl.BlockSpec((bn, H), lambda i: (i, 0)),
            pl.BlockSpec((8, H), lambda i: (0, 0)),
        ],
        out_shape=[
            jax.ShapeDtypeStruct((NP, H), jnp.float32),
            jax.ShapeDtypeStruct((NP, H), jnp.float32),
            jax.ShapeDtypeStruct((8, H), jnp.float32),
        ],
    )(acc, acc, rdeg, wb, a2d)


@jax.jit
def kernel(x, edge_index, W, b, a):
    src = edge_index[0].astype(jnp.int32)
    dst = edge_index[1].astype(jnp.int32)

    x_aug = jnp.concatenate(
        [x, jnp.ones((N, 1), jnp.float32), jnp.zeros((N, 15), jnp.float32)], axis=1)
    x_aug = jnp.pad(x_aug, ((0, NP - N), (0, 0)))
    wb = jnp.concatenate([W, b[None, :], jnp.zeros((15, H), jnp.float32)], axis=0)
    perm = jax.random.permutation(jax.random.key(42), N).astype(jnp.int32)
    perm_pad = jnp.pad(perm, (0, NP - N)).reshape(NT, (NP // NT) // CH, CH)
    dst_a = dst.reshape(NSC * NT, (E // (NSC * NT)) // CH, CH)
    src_c = src.reshape(NT, (E // NT) // CH, CH)
    dst_c = dst.reshape(NT, (E // NT) // CH, CH)
    a2d = a.reshape(1, 1)

    deg = _deg_kernel(dst_a)
    y, rdeg = _scale_call(x_aug, deg)
    acc, _unused = _agg_kernel(y, x_aug, rdeg, perm_pad, src_c, dst_c)
    pz, nz, sm = _out_call(acc, rdeg, wb, a2d)

    return (pz[:N], nz[:N], sm[0, :])
